# baseline (speedup 1.0000x reference)
import functools

import jax
import jax.numpy as jnp
from jax import lax
from jax.experimental import pallas as pl
from jax.experimental.pallas import tpu as pltpu
from jax.experimental.pallas import tpu_sc as plsc

N_ROWS = 100000
B = 16384
D = 64
W = 128
L = 16

_info = plsc.get_sparse_core_info()
NC, NS = _info.num_cores, _info.num_subcores
NW = NC * NS
BPW = B // NW
NROUND = 2
RB = BPW // NROUND
NGROUP = RB // L


def _bpr_body(users_hbm, pos_hbm, neg_hbm, uemb_hbm, iemb_hbm,
              rui_hbm, ruj_hbm, loss_hbm,
              uidx_v, pidx_v, nidx_v, suidx_v, spidx_v, snidx_v,
              urows_v, prows_v, nrows_v,
              rui_v, ruj_v, loss_v, sem):
    wid = lax.axis_index("s") * NC + lax.axis_index("c")
    base = wid * BPW
    pltpu.sync_copy(users_hbm.at[pl.ds(base, BPW)], uidx_v)
    pltpu.sync_copy(pos_hbm.at[pl.ds(base, BPW)], pidx_v)
    pltpu.sync_copy(neg_hbm.at[pl.ds(base, BPW)], nidx_v)

    iota = lax.broadcasted_iota(jnp.int32, (L,), 0)
    zero = jnp.zeros((L,), jnp.float32)

    def shift(i, _):
        s = pl.ds(i * L, L)
        suidx_v[s] = uidx_v[s] >> 1
        spidx_v[s] = pidx_v[s] >> 1
        snidx_v[s] = nidx_v[s] >> 1
        return 0

    lax.fori_loop(0, BPW // L, shift, 0)

    def round_body(r, carry):
        l1, l2, l3 = carry
        copies = []
        for j in range(RB // 128):
            src = pl.ds(r * RB + j * 128, 128)
            dst = pl.ds(j * 128, 128)
            copies.append(pltpu.async_copy(uemb_hbm.at[suidx_v.at[src]],
                                           urows_v.at[dst], sem))
            copies.append(pltpu.async_copy(iemb_hbm.at[spidx_v.at[src]],
                                           prows_v.at[dst], sem))
            copies.append(pltpu.async_copy(iemb_hbm.at[snidx_v.at[src]],
                                           nrows_v.at[dst], sem))
        for c in copies:
            c.wait()

        def group(g, carry2):
            l1, l2, l3 = carry2
            rowv = g * L + iota
            out = pl.ds(r * RB + g * L, L)
            ubase = (uidx_v[out] & 1) << 6
            pbase = (pidx_v[out] & 1) << 6
            nbase = (nidx_v[out] & 1) << 6
            rui_a = zero
            rui_b = zero
            ruj_a = zero
            ruj_b = zero
            for d in range(D):
                colv = iota ^ d
                iu = plsc.load_gather(urows_v, [rowv, ubase | colv])
                ip = plsc.load_gather(prows_v, [rowv, pbase | colv])
                iv = plsc.load_gather(nrows_v, [rowv, nbase | colv])
                if d % 2 == 0:
                    rui_a = rui_a + iu * ip
                    ruj_a = ruj_a + iu * iv
                else:
                    rui_b = rui_b + iu * ip
                    ruj_b = ruj_b + iu * iv
                l1 = l1 + iu * iu
                l2 = l2 + ip * ip
                l3 = l3 + iv * iv
            rui_v[out] = rui_a + rui_b
            ruj_v[out] = ruj_a + ruj_b
            return (l1, l2, l3)

        return lax.fori_loop(0, NGROUP, group, (l1, l2, l3))

    carry = (zero, zero, zero)
    for r in range(NROUND):
        carry = round_body(r, carry)
    l1, l2, l3 = carry
    loss_v[...] = l1 + l2 + l3

    pltpu.sync_copy(rui_v, rui_hbm.at[pl.ds(base, BPW)])
    pltpu.sync_copy(ruj_v, ruj_hbm.at[pl.ds(base, BPW)])
    pltpu.sync_copy(loss_v, loss_hbm.at[wid])


@jax.jit
def _bpr_sc(users, pos_items, neg_items, user_emb, item_emb):
    mesh = plsc.VectorSubcoreMesh(core_axis_name="c", subcore_axis_name="s")
    k = functools.partial(
        pl.kernel,
        mesh=mesh,
        compiler_params=pltpu.CompilerParams(needs_layout_passes=False,
                                             use_tc_tiling_on_sc=True),
        out_type=[
            jax.ShapeDtypeStruct((B,), jnp.float32),
            jax.ShapeDtypeStruct((B,), jnp.float32),
            jax.ShapeDtypeStruct((NW, L), jnp.float32),
        ],
        scratch_types=[
            pltpu.VMEM((BPW,), jnp.int32),
            pltpu.VMEM((BPW,), jnp.int32),
            pltpu.VMEM((BPW,), jnp.int32),
            pltpu.VMEM((BPW,), jnp.int32),
            pltpu.VMEM((BPW,), jnp.int32),
            pltpu.VMEM((BPW,), jnp.int32),
            pltpu.VMEM((RB, W), jnp.float32),
            pltpu.VMEM((RB, W), jnp.float32),
            pltpu.VMEM((RB, W), jnp.float32),
            pltpu.VMEM((BPW,), jnp.float32),
            pltpu.VMEM((BPW,), jnp.float32),
            pltpu.VMEM((L,), jnp.float32),
            pltpu.SemaphoreType.DMA,
        ],
    )(_bpr_body)
    upad = user_emb.reshape(N_ROWS // 2, W)
    ipad = item_emb.reshape(N_ROWS // 2, W)
    rui, ruj, loss_parts = k(users.astype(jnp.int32), pos_items.astype(jnp.int32),
                             neg_items.astype(jnp.int32), upad, ipad)
    return (rui.reshape(B, 1), ruj.reshape(B, 1), jnp.sum(loss_parts))


def kernel(users, pos_items, neg_items, user_emb, item_emb):
    return _bpr_sc(users, pos_items, neg_items, user_emb, item_emb)


# speedup vs baseline: 1.5956x; 1.5956x over previous
import functools

import jax
import jax.numpy as jnp
from jax import lax
from jax.experimental import pallas as pl
from jax.experimental.pallas import tpu as pltpu
from jax.experimental.pallas import tpu_sc as plsc

N_ROWS = 100000
B = 16384
D = 64
W = 128
L = 16

_info = plsc.get_sparse_core_info()
NC, NS = _info.num_cores, _info.num_subcores
NW = NC * NS
BPW = B // NW
NROUND = 2
RB = BPW // NROUND
NGROUP = RB // L


def _bpr_body(users_hbm, pos_hbm, neg_hbm, uemb_hbm, iemb_hbm,
              rui_hbm, ruj_hbm, loss_hbm,
              uidx_v, pidx_v, nidx_v, suidx_v, spidx_v, snidx_v,
              urows_v, prows_v, nrows_v,
              rui_v, ruj_v, loss_v, sem):
    wid = lax.axis_index("s") * NC + lax.axis_index("c")
    base = wid * BPW
    pltpu.sync_copy(users_hbm.at[pl.ds(base, BPW)], uidx_v)
    pltpu.sync_copy(pos_hbm.at[pl.ds(base, BPW)], pidx_v)
    pltpu.sync_copy(neg_hbm.at[pl.ds(base, BPW)], nidx_v)

    iota = lax.broadcasted_iota(jnp.int32, (L,), 0)
    zero = jnp.zeros((L,), jnp.float32)

    def shift(i, _):
        s = pl.ds(i * L, L)
        u, p, n = uidx_v[s], pidx_v[s], nidx_v[s]
        suidx_v[s] = ((u >> 13) << 12) | (u & 4095)
        spidx_v[s] = ((p >> 13) << 12) | (p & 4095)
        snidx_v[s] = ((n >> 13) << 12) | (n & 4095)
        return 0

    lax.fori_loop(0, BPW // L, shift, 0)

    def round_body(r, carry):
        l1, l2, l3 = carry
        copies = []
        for j in range(RB // 128):
            src = pl.ds(r * RB + j * 128, 128)
            dst = pl.ds(j * 128, 128)
            copies.append(pltpu.async_copy(uemb_hbm.at[suidx_v.at[src]],
                                           urows_v.at[dst], sem))
            copies.append(pltpu.async_copy(iemb_hbm.at[spidx_v.at[src]],
                                           prows_v.at[dst], sem))
            copies.append(pltpu.async_copy(iemb_hbm.at[snidx_v.at[src]],
                                           nrows_v.at[dst], sem))
        for c in copies:
            c.wait()

        def group(g, carry2):
            l1, l2, l3 = carry2
            rowv = g * L + iota
            out = pl.ds(r * RB + g * L, L)
            ubase = ((uidx_v[out] >> 12) & 1) << 6
            pbase = ((pidx_v[out] >> 12) & 1) << 6
            nbase = ((nidx_v[out] >> 12) & 1) << 6
            rui_a = zero
            rui_b = zero
            ruj_a = zero
            ruj_b = zero
            for d in range(D):
                colv = iota ^ d
                iu = plsc.load_gather(urows_v, [rowv, ubase | colv])
                ip = plsc.load_gather(prows_v, [rowv, pbase | colv])
                iv = plsc.load_gather(nrows_v, [rowv, nbase | colv])
                if d % 2 == 0:
                    rui_a = rui_a + iu * ip
                    ruj_a = ruj_a + iu * iv
                else:
                    rui_b = rui_b + iu * ip
                    ruj_b = ruj_b + iu * iv
                l1 = l1 + iu * iu
                l2 = l2 + ip * ip
                l3 = l3 + iv * iv
            rui_v[out] = rui_a + rui_b
            ruj_v[out] = ruj_a + ruj_b
            return (l1, l2, l3)

        return lax.fori_loop(0, NGROUP, group, (l1, l2, l3))

    carry = (zero, zero, zero)
    for r in range(NROUND):
        carry = round_body(r, carry)
    l1, l2, l3 = carry
    loss_v[...] = l1 + l2 + l3

    pltpu.sync_copy(rui_v, rui_hbm.at[pl.ds(base, BPW)])
    pltpu.sync_copy(ruj_v, ruj_hbm.at[pl.ds(base, BPW)])
    pltpu.sync_copy(loss_v, loss_hbm.at[wid])


_NBT = 8192
_HBT = _NBT // 2
_NBLK = (N_ROWS + _NBT - 1) // _NBT
_PACKED_ROWS = _NBLK * _HBT


def _pack_body(xt_ref, out_ref):
    y = xt_ref[...].T
    out_ref[...] = jnp.concatenate([y[:_HBT], y[_HBT:]], axis=1)


def _pack_pairs(table_t):
    return pl.pallas_call(
        _pack_body,
        grid=(_NBLK,),
        in_specs=[pl.BlockSpec((D, _NBT), lambda i: (0, i))],
        out_specs=pl.BlockSpec((_HBT, W), lambda i: (i, 0)),
        out_shape=jax.ShapeDtypeStruct((_PACKED_ROWS, W), jnp.float32),
    )(table_t)


@jax.jit
def _bpr_sc(users, pos_items, neg_items, user_emb, item_emb):
    mesh = plsc.VectorSubcoreMesh(core_axis_name="c", subcore_axis_name="s")
    k = functools.partial(
        pl.kernel,
        mesh=mesh,
        compiler_params=pltpu.CompilerParams(needs_layout_passes=False,
                                             use_tc_tiling_on_sc=True),
        out_type=[
            jax.ShapeDtypeStruct((B,), jnp.float32),
            jax.ShapeDtypeStruct((B,), jnp.float32),
            jax.ShapeDtypeStruct((NW, L), jnp.float32),
        ],
        scratch_types=[
            pltpu.VMEM((BPW,), jnp.int32),
            pltpu.VMEM((BPW,), jnp.int32),
            pltpu.VMEM((BPW,), jnp.int32),
            pltpu.VMEM((BPW,), jnp.int32),
            pltpu.VMEM((BPW,), jnp.int32),
            pltpu.VMEM((BPW,), jnp.int32),
            pltpu.VMEM((RB, W), jnp.float32),
            pltpu.VMEM((RB, W), jnp.float32),
            pltpu.VMEM((RB, W), jnp.float32),
            pltpu.VMEM((BPW,), jnp.float32),
            pltpu.VMEM((BPW,), jnp.float32),
            pltpu.VMEM((L,), jnp.float32),
            pltpu.SemaphoreType.DMA,
        ],
    )(_bpr_body)
    upad = _pack_pairs(user_emb.T)
    ipad = _pack_pairs(item_emb.T)
    rui, ruj, loss_parts = k(users.astype(jnp.int32), pos_items.astype(jnp.int32),
                             neg_items.astype(jnp.int32), upad, ipad)
    return (rui.reshape(B, 1), ruj.reshape(B, 1), jnp.sum(loss_parts))


def kernel(users, pos_items, neg_items, user_emb, item_emb):
    return _bpr_sc(users, pos_items, neg_items, user_emb, item_emb)
